# Initial kernel scaffold; baseline (speedup 1.0000x reference)
import functools

import jax
import jax.numpy as jnp
import numpy as np
from jax import lax
from jax.experimental import pallas as pl
from jax.experimental.pallas import tpu as pltpu
from jax.experimental.pallas import tpu_sc as plsc

N_NODES = 10000
N_EDGES = 320000
HEADS = 8
DH = 8
HID = 64

NC = 2
NS = 16
NTILES = NC * NS
EDGES_PER_TILE = N_EDGES // NTILES
EDGE_BATCH = 200
N_BATCHES = EDGES_PER_TILE // EDGE_BATCH
ROWS_PER_TILE = N_NODES // NS

_P = np.empty(64, np.int32)
for _h in range(HEADS):
    for _d in range(DH):
        _col = 16 * (_d // 2) + (_h if _d % 2 == 0 else 15 - _h)
        _P[_col] = _h * DH + _d
_P_JNP = jnp.asarray(_P)


def _edge_body(qt, kt, vt, et, src, dst, zwv, zz,
               out_wv, out_z,
               acc_wv, acc_z, srcb, dstb, kb, qb, vb, eb, ob, obz, sem):
    cid = lax.axis_index("c")
    sid = lax.axis_index("s")
    wid = cid * NS + sid

    r0 = sid * ROWS_PER_TILE
    pltpu.sync_copy(zwv.at[pl.ds(r0, ROWS_PER_TILE)],
                    acc_wv.at[pl.ds(r0, ROWS_PER_TILE)])
    pltpu.sync_copy(zz.at[pl.ds(r0, ROWS_PER_TILE)],
                    acc_z.at[pl.ds(r0, ROWS_PER_TILE)])
    plsc.subcore_barrier()

    base0 = wid * EDGES_PER_TILE
    inv_sqrt_dh = np.float32(1.0 / np.sqrt(DH))

    def batch(b, carry):
        base = base0 + b * EDGE_BATCH
        pltpu.sync_copy(src.at[pl.ds(base, EDGE_BATCH)], srcb)
        pltpu.sync_copy(dst.at[pl.ds(base, EDGE_BATCH)], dstb)
        cp_k = pltpu.async_copy(kt.at[srcb], kb, sem)
        cp_q = pltpu.async_copy(qt.at[dstb], qb, sem)
        cp_v = pltpu.async_copy(vt.at[srcb], vb, sem)
        cp_e = pltpu.async_copy(et.at[pl.ds(base, EDGE_BATCH)], eb, sem)
        cp_k.wait()
        cp_q.wait()
        cp_v.wait()
        cp_e.wait()

        def edge(i, carry2):
            t = None
            for j in range(4):
                sl = pl.ds(j * 16, 16)
                kq = kb[i, sl] * qb[i, sl] * eb[i, sl]
                t = kq if t is None else t + kq
            t = t * inv_sqrt_dh
            s = t + lax.rev(t, (0,))
            s = jnp.minimum(jnp.maximum(s, -5.0), 5.0)
            w = jnp.exp(s)
            for j in range(4):
                sl = pl.ds(j * 16, 16)
                ob[i, sl] = vb[i, sl] * w
            obz[i, pl.ds(0, 16)] = w
            return carry2

        lax.fori_loop(0, EDGE_BATCH, edge, 0)
        pltpu.sync_copy(ob, acc_wv.at[dstb], add=True)
        pltpu.sync_copy(obz, acc_z.at[dstb], add=True)
        return carry

    lax.fori_loop(0, N_BATCHES, batch, 0)
    plsc.subcore_barrier()

    pltpu.sync_copy(acc_wv.at[pl.ds(r0, ROWS_PER_TILE)],
                    out_wv.at[cid, pl.ds(r0, ROWS_PER_TILE)])
    pltpu.sync_copy(acc_z.at[pl.ds(r0, ROWS_PER_TILE)],
                    out_z.at[cid, pl.ds(r0, ROWS_PER_TILE)])


_edge_call = functools.partial(
    pl.kernel,
    out_type=(
        jax.ShapeDtypeStruct((NC, N_NODES, 64), jnp.float32),
        jax.ShapeDtypeStruct((NC, N_NODES, 16), jnp.float32),
    ),
    mesh=plsc.VectorSubcoreMesh(core_axis_name="c", subcore_axis_name="s"),
    scratch_types=[
        pltpu.VMEM_SHARED((N_NODES, 64), jnp.float32),
        pltpu.VMEM_SHARED((N_NODES, 16), jnp.float32),
        pltpu.VMEM((EDGE_BATCH,), jnp.int32),
        pltpu.VMEM((EDGE_BATCH,), jnp.int32),
        pltpu.VMEM((EDGE_BATCH, 64), jnp.float32),
        pltpu.VMEM((EDGE_BATCH, 64), jnp.float32),
        pltpu.VMEM((EDGE_BATCH, 64), jnp.float32),
        pltpu.VMEM((EDGE_BATCH, 64), jnp.float32),
        pltpu.VMEM((EDGE_BATCH, 64), jnp.float32),
        pltpu.VMEM((EDGE_BATCH, 16), jnp.float32),
        pltpu.SemaphoreType.DMA,
    ],
)(_edge_body)


def _edge_phase(qt, kt, vt, et, src, dst):
    zwv = jnp.zeros((N_NODES, 64), jnp.float32)
    zz = jnp.zeros((N_NODES, 16), jnp.float32)
    return _edge_call(qt, kt, vt, et, src, dst, zwv, zz)


def _dot(a, b):
    return jnp.dot(a, b, preferred_element_type=jnp.float32)


def _bn_fwd(x, g, b):
    m = jnp.mean(x, axis=0, keepdims=True)
    v = jnp.mean((x - m) ** 2, axis=0, keepdims=True)
    return g * (x - m) * jax.lax.rsqrt(v + 1e-5) + b


def _prep_body(h_r, wh_r, bh_r, wq_r, wk_r, wv_r, hh_o, qt_o, kt_o, vt_o):
    hh = _dot(h_r[...], wh_r[...]) + bh_r[...]
    hh_o[...] = hh
    qt_o[...] = _dot(hh, wq_r[...])
    kt_o[...] = _dot(hh, wk_r[...])
    vt_o[...] = _dot(hh, wv_r[...])


def _prep(h, wh, bh, wq_t, wk_t, wv_t):
    f = jax.ShapeDtypeStruct
    return pl.pallas_call(
        _prep_body,
        out_shape=(
            f((N_NODES, HID), jnp.float32),
            f((N_NODES, HID), jnp.float32),
            f((N_NODES, HID), jnp.float32),
            f((N_NODES, HID), jnp.float32),
        ),
    )(h, wh, bh, wq_t, wk_t, wv_t)


_EBLK = 3200


def _eproj_body(e_r, m0_r, c0_r, m1_r, c1_r, et0_o, et1_o):
    e_blk = e_r[...]
    et0_o[...] = _dot(e_blk, m0_r[...]) + c0_r[...]
    et1_o[...] = _dot(e_blk, m1_r[...]) + c1_r[...]


def _eproj(e, m0, c0, m1, c1):
    f = jax.ShapeDtypeStruct
    grid = N_EDGES // _EBLK
    return pl.pallas_call(
        _eproj_body,
        grid=(grid,),
        in_specs=[
            pl.BlockSpec((_EBLK, 16), lambda i: (i, 0)),
            pl.BlockSpec((16, 64), lambda i: (0, 0)),
            pl.BlockSpec((1, 64), lambda i: (0, 0)),
            pl.BlockSpec((16, 64), lambda i: (0, 0)),
            pl.BlockSpec((1, 64), lambda i: (0, 0)),
        ],
        out_specs=(
            pl.BlockSpec((_EBLK, 64), lambda i: (i, 0)),
            pl.BlockSpec((_EBLK, 64), lambda i: (i, 0)),
        ),
        out_shape=(
            f((N_EDGES, 64), jnp.float32),
            f((N_EDGES, 64), jnp.float32),
        ),
    )(e, m0, c0, m1, c1)


def _attn_out(part_wv, part_z, wo_p, bo):
    wv = part_wv[0] + part_wv[1]
    z16 = part_z[0] + part_z[1]
    zc = jnp.concatenate([z16, z16, z16, z16], axis=1)
    ha = wv / (zc + 1e-6)
    return _dot(ha, wo_p) + bo


def _ffn(x, w1, b1, w2, b2):
    t = jax.nn.relu(_dot(x, w1) + b1)
    return _dot(t, w2) + b2


def _post0_body(pwv_r, pz_r, hh0_r, wo_r, bo_r, g1_r, be1_r, w1_r, b1_r,
                w2_r, b2_r, g2_r, be2_r, wq_r, wk_r, wv_r,
                qt_o, kt_o, vt_o):
    hh = _attn_out(pwv_r[...], pz_r[...], wo_r[...], bo_r[...])
    hh = hh0_r[...] + hh
    hh = _bn_fwd(hh, g1_r[...], be1_r[...])
    hh = hh + _ffn(hh, w1_r[...], b1_r[...], w2_r[...], b2_r[...])
    hh = _bn_fwd(hh, g2_r[...], be2_r[...])
    qt_o[...] = _dot(hh, wq_r[...])
    kt_o[...] = _dot(hh, wk_r[...])
    vt_o[...] = _dot(hh, wv_r[...])


def _post0(pwv, pz, hh0, wo_p, bo, g1, be1, w1, b1, w2, b2, g2, be2,
           wq1_t, wk1_t, wv1_t):
    f = jax.ShapeDtypeStruct
    return pl.pallas_call(
        _post0_body,
        out_shape=(
            f((N_NODES, HID), jnp.float32),
            f((N_NODES, HID), jnp.float32),
            f((N_NODES, HID), jnp.float32),
        ),
    )(pwv, pz, hh0, wo_p, bo, g1, be1, w1, b1, w2, b2, g2, be2,
      wq1_t, wk1_t, wv1_t)


def _post1_body(pwv_r, pz_r, wo_r, bo_r, g1_r, be1_r, w1_r, b1_r,
                w2_r, b2_r, g2_r, be2_r, mw0_r, mb0_r, mw1_r, mb1_r,
                mw2_r, mb2_r, y_o):
    hh = _attn_out(pwv_r[...], pz_r[...], wo_r[...], bo_r[...])
    hh = _bn_fwd(hh, g1_r[...], be1_r[...])
    hh = _ffn(hh, w1_r[...], b1_r[...], w2_r[...], b2_r[...])
    hh = _bn_fwd(hh, g2_r[...], be2_r[...])
    hg = jnp.mean(hh, axis=0, keepdims=True)
    y = jax.nn.relu(_dot(hg, mw0_r[...]) + mb0_r[...])
    y = jax.nn.relu(_dot(y, mw1_r[...]) + mb1_r[...])
    y_o[...] = _dot(y, mw2_r[...]) + mb2_r[...]


def _post1(pwv, pz, wo_p, bo, g1, be1, w1, b1, w2, b2, g2, be2,
           mw0, mb0, mw1, mb1, mw2, mb2):
    return pl.pallas_call(
        _post1_body,
        out_shape=jax.ShapeDtypeStruct((1, 3), jnp.float32),
    )(pwv, pz, wo_p, bo, g1, be1, w1, b1, w2, b2, g2, be2,
      mw0, mb0, mw1, mb1, mw2, mb2)


def _row(v):
    return v.reshape(1, -1)


def kernel(h, e, edge_index, params):
    p0 = params['layer0']
    p1 = params['layer1']

    wq0 = p0['Wq'][:, _P_JNP]
    wk0 = p0['Wk'][:, _P_JNP]
    wv0 = p0['Wv'][:, _P_JNP]
    wq1 = p1['Wq'][:, _P_JNP]
    wk1 = p1['Wk'][:, _P_JNP]
    wv1 = p1['Wv'][:, _P_JNP]
    wo0 = p0['Wo'][_P_JNP, :]
    wo1 = p1['Wo'][_P_JNP, :]
    m0 = (params['We_emb'] @ p0['We'])[:, _P_JNP]
    c0 = _row((params['be_emb'] @ p0['We'])[_P_JNP])
    m1 = (params['We_emb'] @ p1['We'])[:, _P_JNP]
    c1 = _row((params['be_emb'] @ p1['We'])[_P_JNP])

    src = edge_index[0]
    dst = edge_index[1]

    hh0, qt0, kt0, vt0 = _prep(h, params['Wh'], _row(params['bh']),
                               wq0, wk0, wv0)
    et0, et1 = _eproj(e, m0, c0, m1, c1)

    pwv0, pz0 = _edge_phase(qt0, kt0, vt0, et0, src, dst)
    qt1, kt1, vt1 = _post0(
        pwv0, pz0, hh0, wo0, _row(p0['bo']),
        _row(p0['bn1_g']), _row(p0['bn1_b']),
        p0['W1'], _row(p0['b1']), p0['W2'], _row(p0['b2']),
        _row(p0['bn2_g']), _row(p0['bn2_b']),
        wq1, wk1, wv1)

    pwv1, pz1 = _edge_phase(qt1, kt1, vt1, et1, src, dst)
    y = _post1(
        pwv1, pz1, wo1, _row(p1['bo']),
        _row(p1['bn1_g']), _row(p1['bn1_b']),
        p1['W1'], _row(p1['b1']), p1['W2'], _row(p1['b2']),
        _row(p1['bn2_g']), _row(p1['bn2_b']),
        params['mlp_W0'], _row(params['mlp_b0']),
        params['mlp_W1'], _row(params['mlp_b1']),
        params['mlp_W2'], _row(params['mlp_b2']))
    return y


# speedup vs baseline: 60.0889x; 60.0889x over previous
import functools

import jax
import jax.numpy as jnp
import numpy as np
from jax import lax
from jax.experimental import pallas as pl
from jax.experimental.pallas import tpu as pltpu
from jax.experimental.pallas import tpu_sc as plsc

N_NODES = 10000
N_EDGES = 320000
HEADS = 8
DH = 8
HID = 64

NC = 2
NS = 16
NTILES = NC * NS
EDGES_PER_TILE = N_EDGES // NTILES
EDGE_BATCH = 80
N_BATCHES = EDGES_PER_TILE // EDGE_BATCH
N_PAD = 10240
ROWS_PER_TILE = N_PAD // NS

_P = np.empty(64, np.int32)
for _h in range(HEADS):
    for _d in range(DH):
        _col = 16 * (_d // 2) + (_h if _d % 2 == 0 else 15 - _h)
        _P[_col] = _h * DH + _d


def _edge_body(qt, kvt, et, src, dst, zac,
               out_acc,
               acc, srcb, dstb, kvb, qb, eb, ob, sem):
    cid = lax.axis_index("c")
    sid = lax.axis_index("s")
    wid = cid * NS + sid

    r0 = sid * ROWS_PER_TILE
    pltpu.sync_copy(zac.at[pl.ds(r0, ROWS_PER_TILE)],
                    acc.at[pl.ds(r0, ROWS_PER_TILE)])

    def zrow(i, c):
        for j in range(5, 8):
            ob[i, pl.ds(j * 16, 16)] = jnp.zeros((16,), jnp.float32)
        return c

    lax.fori_loop(0, EDGE_BATCH, zrow, 0)
    plsc.subcore_barrier()

    base0 = wid * EDGES_PER_TILE
    inv_sqrt_dh = np.float32(1.0 / np.sqrt(DH))

    def batch(b, carry):
        base = base0 + b * EDGE_BATCH
        pltpu.sync_copy(src.at[pl.ds(base, EDGE_BATCH)], srcb)
        pltpu.sync_copy(dst.at[pl.ds(base, EDGE_BATCH)], dstb)
        cp_kv = pltpu.async_copy(kvt.at[srcb], kvb, sem)
        cp_q = pltpu.async_copy(qt.at[dstb], qb, sem)
        cp_e = pltpu.async_copy(et.at[pl.ds(base, EDGE_BATCH)], eb, sem)
        cp_kv.wait()
        cp_q.wait()
        cp_e.wait()

        def edge(i, carry2):
            t = None
            for j in range(4):
                sl = pl.ds(j * 16, 16)
                kq = kvb[i, sl] * qb[i, sl] * eb[i, sl]
                t = kq if t is None else t + kq
            t = t * inv_sqrt_dh
            s = t + lax.rev(t, (0,))
            s = jnp.minimum(jnp.maximum(s, -5.0), 5.0)
            w = jnp.exp(s)
            for j in range(4):
                ob[i, pl.ds(j * 16, 16)] = kvb[i, pl.ds(64 + j * 16, 16)] * w
            ob[i, pl.ds(64, 16)] = w
            return carry2

        lax.fori_loop(0, EDGE_BATCH, edge, 0)
        pltpu.sync_copy(ob, acc.at[dstb], add=True)
        return carry

    lax.fori_loop(0, N_BATCHES, batch, 0)
    plsc.subcore_barrier()

    pltpu.sync_copy(acc.at[pl.ds(r0, ROWS_PER_TILE)],
                    out_acc.at[cid, pl.ds(r0, ROWS_PER_TILE)])


@functools.cache
def _edge_call():
    return pl.kernel(
        _edge_body,
        out_type=jax.ShapeDtypeStruct((NC, N_PAD, 128), jnp.float32),
        mesh=plsc.VectorSubcoreMesh(core_axis_name="c", subcore_axis_name="s"),
        scratch_types=[
            pltpu.VMEM_SHARED((N_PAD, 128), jnp.float32),
            pltpu.VMEM((EDGE_BATCH,), jnp.int32),
            pltpu.VMEM((EDGE_BATCH,), jnp.int32),
            pltpu.VMEM((EDGE_BATCH, 128), jnp.float32),
            pltpu.VMEM((EDGE_BATCH, 128), jnp.float32),
            pltpu.VMEM((EDGE_BATCH, 64), jnp.float32),
            pltpu.VMEM((EDGE_BATCH, 128), jnp.float32),
            pltpu.SemaphoreType.DMA,
        ],
    )


def _edge_phase(qt, kvt, et, src, dst):
    zac = jnp.zeros((N_PAD, 128), jnp.float32)
    return _edge_call()(qt, kvt, et, src, dst, zac)


def _dot(a, b):
    return jnp.dot(a, b, preferred_element_type=jnp.float32)


def _bn_fwd(x, g, b):
    m = jnp.mean(x, axis=0, keepdims=True)
    v = jnp.mean((x - m) ** 2, axis=0, keepdims=True)
    return g * (x - m) * jax.lax.rsqrt(v + 1e-5) + b


def _prep_body(h_r, wh_r, bh_r, wq_r, wkv_r, hh_o, qt_o, kvt_o):
    hh = _dot(h_r[...], wh_r[...]) + bh_r[...]
    hh_o[...] = hh
    q = _dot(hh, wq_r[...])
    qt_o[...] = jnp.concatenate([q, jnp.zeros_like(q)], axis=1)
    kvt_o[...] = _dot(hh, wkv_r[...])


def _prep(h, wh, bh, wq_t, wkv_t):
    f = jax.ShapeDtypeStruct
    return pl.pallas_call(
        _prep_body,
        out_shape=(
            f((N_NODES, HID), jnp.float32),
            f((N_NODES, 128), jnp.float32),
            f((N_NODES, 128), jnp.float32),
        ),
    )(h, wh, bh, wq_t, wkv_t)


_EBLK = 3200


def _eproj_body(e_r, m0_r, c0_r, m1_r, c1_r, et0_o, et1_o):
    e_blk = e_r[...]
    et0_o[...] = _dot(e_blk, m0_r[...]) + c0_r[...]
    et1_o[...] = _dot(e_blk, m1_r[...]) + c1_r[...]


def _eproj(e, m0, c0, m1, c1):
    f = jax.ShapeDtypeStruct
    grid = N_EDGES // _EBLK
    return pl.pallas_call(
        _eproj_body,
        grid=(grid,),
        in_specs=[
            pl.BlockSpec((_EBLK, 16), lambda i: (i, 0)),
            pl.BlockSpec((16, 64), lambda i: (0, 0)),
            pl.BlockSpec((1, 64), lambda i: (0, 0)),
            pl.BlockSpec((16, 64), lambda i: (0, 0)),
            pl.BlockSpec((1, 64), lambda i: (0, 0)),
        ],
        out_specs=(
            pl.BlockSpec((_EBLK, 64), lambda i: (i, 0)),
            pl.BlockSpec((_EBLK, 64), lambda i: (i, 0)),
        ),
        out_shape=(
            f((N_EDGES, 64), jnp.float32),
            f((N_EDGES, 64), jnp.float32),
        ),
    )(e, m0, c0, m1, c1)


def _attn_out(part, wo_p, bo):
    p = (part[0] + part[1])[:N_NODES]
    wv = p[:, 0:64]
    z16 = p[:, 64:80]
    zc = jnp.concatenate([z16, z16, z16, z16], axis=1)
    ha = wv / (zc + 1e-6)
    return _dot(ha, wo_p) + bo


def _ffn(x, w1, b1, w2, b2):
    t = jax.nn.relu(_dot(x, w1) + b1)
    return _dot(t, w2) + b2


def _post0_body(part_r, hh0_r, wo_r, bo_r, g1_r, be1_r, w1_r, b1_r,
                w2_r, b2_r, g2_r, be2_r, wq_r, wkv_r,
                qt_o, kvt_o):
    hh = _attn_out(part_r[...], wo_r[...], bo_r[...])
    hh = hh0_r[...] + hh
    hh = _bn_fwd(hh, g1_r[...], be1_r[...])
    hh = hh + _ffn(hh, w1_r[...], b1_r[...], w2_r[...], b2_r[...])
    hh = _bn_fwd(hh, g2_r[...], be2_r[...])
    q = _dot(hh, wq_r[...])
    qt_o[...] = jnp.concatenate([q, jnp.zeros_like(q)], axis=1)
    kvt_o[...] = _dot(hh, wkv_r[...])


def _post0(part, hh0, wo_p, bo, g1, be1, w1, b1, w2, b2, g2, be2,
           wq1_t, wkv1_t):
    f = jax.ShapeDtypeStruct
    return pl.pallas_call(
        _post0_body,
        out_shape=(
            f((N_NODES, 128), jnp.float32),
            f((N_NODES, 128), jnp.float32),
        ),
    )(part, hh0, wo_p, bo, g1, be1, w1, b1, w2, b2, g2, be2,
      wq1_t, wkv1_t)


def _post1_body(part_r, wo_r, bo_r, g1_r, be1_r, w1_r, b1_r,
                w2_r, b2_r, g2_r, be2_r, mw0_r, mb0_r, mw1_r, mb1_r,
                mw2_r, mb2_r, y_o):
    hh = _attn_out(part_r[...], wo_r[...], bo_r[...])
    hh = _bn_fwd(hh, g1_r[...], be1_r[...])
    hh = _ffn(hh, w1_r[...], b1_r[...], w2_r[...], b2_r[...])
    hh = _bn_fwd(hh, g2_r[...], be2_r[...])
    hg = jnp.mean(hh, axis=0, keepdims=True)
    y = jax.nn.relu(_dot(hg, mw0_r[...]) + mb0_r[...])
    y = jax.nn.relu(_dot(y, mw1_r[...]) + mb1_r[...])
    y_o[...] = _dot(y, mw2_r[...]) + mb2_r[...]


def _post1(part, wo_p, bo, g1, be1, w1, b1, w2, b2, g2, be2,
           mw0, mb0, mw1, mb1, mw2, mb2):
    return pl.pallas_call(
        _post1_body,
        out_shape=jax.ShapeDtypeStruct((1, 3), jnp.float32),
    )(part, wo_p, bo, g1, be1, w1, b1, w2, b2, g2, be2,
      mw0, mb0, mw1, mb1, mw2, mb2)


def _row(v):
    return v.reshape(1, -1)


def kernel(h, e, edge_index, params):
    p0 = params['layer0']
    p1 = params['layer1']

    wq0 = p0['Wq'][:, _P]
    wk0 = p0['Wk'][:, _P]
    wv0 = p0['Wv'][:, _P]
    wq1 = p1['Wq'][:, _P]
    wk1 = p1['Wk'][:, _P]
    wv1 = p1['Wv'][:, _P]
    wo0 = p0['Wo'][_P, :]
    wo1 = p1['Wo'][_P, :]
    m0 = (params['We_emb'] @ p0['We'])[:, _P]
    c0 = _row((params['be_emb'] @ p0['We'])[_P])
    m1 = (params['We_emb'] @ p1['We'])[:, _P]
    c1 = _row((params['be_emb'] @ p1['We'])[_P])

    src = edge_index[0]
    dst = edge_index[1]

    wkv0 = jnp.concatenate([wk0, wv0], axis=1)
    wkv1 = jnp.concatenate([wk1, wv1], axis=1)

    hh0, qt0, kvt0 = _prep(h, params['Wh'], _row(params['bh']), wq0, wkv0)
    et0, et1 = _eproj(e, m0, c0, m1, c1)

    part0 = _edge_phase(qt0, kvt0, et0, src, dst)
    qt1, kvt1 = _post0(
        part0, hh0, wo0, _row(p0['bo']),
        _row(p0['bn1_g']), _row(p0['bn1_b']),
        p0['W1'], _row(p0['b1']), p0['W2'], _row(p0['b2']),
        _row(p0['bn2_g']), _row(p0['bn2_b']),
        wq1, wkv1)

    part1 = _edge_phase(qt1, kvt1, et1, src, dst)
    y = _post1(
        part1, wo1, _row(p1['bo']),
        _row(p1['bn1_g']), _row(p1['bn1_b']),
        p1['W1'], _row(p1['b1']), p1['W2'], _row(p1['b2']),
        _row(p1['bn2_g']), _row(p1['bn2_b']),
        params['mlp_W0'], _row(params['mlp_b0']),
        params['mlp_W1'], _row(params['mlp_b1']),
        params['mlp_W2'], _row(params['mlp_b2']))
    return y


# speedup vs baseline: 71.3609x; 1.1876x over previous
import functools

import jax
import jax.numpy as jnp
import numpy as np
from jax import lax
from jax.experimental import pallas as pl
from jax.experimental.pallas import tpu as pltpu
from jax.experimental.pallas import tpu_sc as plsc

N_NODES = 10000
N_EDGES = 320000
HEADS = 8
DH = 8
HID = 64

NC = 2
NS = 16
NTILES = NC * NS
EDGES_PER_TILE = N_EDGES // NTILES
EDGE_BATCH = 40
N_BATCHES = EDGES_PER_TILE // EDGE_BATCH
N_PAD = 10240
ROWS_PER_TILE = N_PAD // NS

_P = np.empty(64, np.int32)
for _h in range(HEADS):
    for _d in range(DH):
        _col = 16 * (_d // 2) + (_h if _d % 2 == 0 else 15 - _h)
        _P[_col] = _h * DH + _d


def _edge_body(qt, kvt, et, src, dst, zac,
               out_acc,
               acc, srcb, dstb, kvb, qb, eb, ob, semA, semB):
    cid = lax.axis_index("c")
    sid = lax.axis_index("s")
    wid = cid * NS + sid

    r0 = sid * ROWS_PER_TILE
    pltpu.sync_copy(zac.at[pl.ds(r0, ROWS_PER_TILE)],
                    acc.at[pl.ds(r0, ROWS_PER_TILE)])

    def zrow(i, c):
        for j in range(5, 8):
            ob[i, pl.ds(j * 16, 16)] = jnp.zeros((16,), jnp.float32)
        return c

    lax.fori_loop(0, EDGE_BATCH, zrow, 0)
    plsc.subcore_barrier()

    base0 = wid * EDGES_PER_TILE
    inv_sqrt_dh = np.float32(1.0 / np.sqrt(DH))

    def fire(b, buf, sem):
        base = base0 + b * EDGE_BATCH
        pltpu.sync_copy(src.at[pl.ds(base, EDGE_BATCH)], srcb.at[buf])
        pltpu.sync_copy(dst.at[pl.ds(base, EDGE_BATCH)], dstb.at[buf])
        pltpu.async_copy(kvt.at[srcb.at[buf]], kvb.at[buf], sem)
        pltpu.async_copy(qt.at[dstb.at[buf]], qb.at[buf], sem)
        pltpu.async_copy(et.at[pl.ds(base, EDGE_BATCH)], eb.at[buf], sem)

    def drain(buf, sem):
        pltpu.make_async_copy(kvt.at[pl.ds(0, EDGE_BATCH)], kvb.at[buf],
                              sem).wait()
        pltpu.make_async_copy(qt.at[pl.ds(0, EDGE_BATCH)], qb.at[buf],
                              sem).wait()
        pltpu.make_async_copy(et.at[pl.ds(0, EDGE_BATCH)], eb.at[buf],
                              sem).wait()

    def compute(buf):
        def edge(i, carry2):
            t = None
            for j in range(4):
                sl = pl.ds(j * 16, 16)
                kq = kvb[buf, i, sl] * qb[buf, i, sl] * eb[buf, i, sl]
                t = kq if t is None else t + kq
            t = t * inv_sqrt_dh
            s = t + lax.rev(t, (0,))
            s = jnp.minimum(jnp.maximum(s, -5.0), 5.0)
            w = jnp.exp(s)
            for j in range(4):
                ob[i, pl.ds(j * 16, 16)] = kvb[buf, i, pl.ds(64 + j * 16, 16)] * w
            ob[i, pl.ds(64, 16)] = w
            return carry2

        lax.fori_loop(0, EDGE_BATCH, edge, 0)
        pltpu.sync_copy(ob, acc.at[dstb.at[buf]], add=True)

    fire(0, 0, semA)

    def pair(g, carry):
        b0 = 2 * g

        @pl.when(b0 + 1 < N_BATCHES)
        def _():
            fire(b0 + 1, 1, semB)

        drain(0, semA)
        compute(0)

        @pl.when(b0 + 2 < N_BATCHES)
        def _():
            fire(b0 + 2, 0, semA)

        @pl.when(b0 + 1 < N_BATCHES)
        def _():
            drain(1, semB)
            compute(1)

        return carry

    lax.fori_loop(0, (N_BATCHES + 1) // 2, pair, 0)
    plsc.subcore_barrier()

    pltpu.sync_copy(acc.at[pl.ds(r0, ROWS_PER_TILE)],
                    out_acc.at[cid, pl.ds(r0, ROWS_PER_TILE)])


@functools.cache
def _edge_call():
    return pl.kernel(
        _edge_body,
        out_type=jax.ShapeDtypeStruct((NC, N_PAD, 128), jnp.float32),
        mesh=plsc.VectorSubcoreMesh(core_axis_name="c", subcore_axis_name="s"),
        scratch_types=[
            pltpu.VMEM_SHARED((N_PAD, 128), jnp.float32),
            pltpu.VMEM((2, EDGE_BATCH), jnp.int32),
            pltpu.VMEM((2, EDGE_BATCH), jnp.int32),
            pltpu.VMEM((2, EDGE_BATCH, 128), jnp.float32),
            pltpu.VMEM((2, EDGE_BATCH, 128), jnp.float32),
            pltpu.VMEM((2, EDGE_BATCH, 64), jnp.float32),
            pltpu.VMEM((EDGE_BATCH, 128), jnp.float32),
            pltpu.SemaphoreType.DMA,
            pltpu.SemaphoreType.DMA,
        ],
    )


def _edge_phase(qt, kvt, et, src, dst):
    zac = jnp.zeros((N_PAD, 128), jnp.float32)
    return _edge_call()(qt, kvt, et, src, dst, zac)


def _dot(a, b):
    return jnp.dot(a, b, preferred_element_type=jnp.float32)


def _bn_fwd(x, g, b):
    m = jnp.mean(x, axis=0, keepdims=True)
    v = jnp.mean((x - m) ** 2, axis=0, keepdims=True)
    return g * (x - m) * jax.lax.rsqrt(v + 1e-5) + b


def _prep_body(h_r, wh_r, bh_r, wq_r, wkv_r, hh_o, qt_o, kvt_o):
    hh = _dot(h_r[...], wh_r[...]) + bh_r[...]
    hh_o[...] = hh
    q = _dot(hh, wq_r[...])
    qt_o[...] = jnp.concatenate([q, jnp.zeros_like(q)], axis=1)
    kvt_o[...] = _dot(hh, wkv_r[...])


def _prep(h, wh, bh, wq_t, wkv_t):
    f = jax.ShapeDtypeStruct
    return pl.pallas_call(
        _prep_body,
        out_shape=(
            f((N_NODES, HID), jnp.float32),
            f((N_NODES, 128), jnp.float32),
            f((N_NODES, 128), jnp.float32),
        ),
    )(h, wh, bh, wq_t, wkv_t)


_EBLK = 3200


def _eproj_body(e_r, m_r, c_r, et_o):
    et_o[...] = _dot(e_r[...], m_r[...]) + c_r[...]


def _eproj(e, m, c):
    return pl.pallas_call(
        _eproj_body,
        grid=(N_EDGES // _EBLK,),
        in_specs=[
            pl.BlockSpec((_EBLK, 16), lambda i: (i, 0)),
            pl.BlockSpec((16, 64), lambda i: (0, 0)),
            pl.BlockSpec((1, 64), lambda i: (0, 0)),
        ],
        out_specs=pl.BlockSpec((_EBLK, 64), lambda i: (i, 0)),
        out_shape=jax.ShapeDtypeStruct((N_EDGES, 64), jnp.float32),
    )(e, m, c)


def _attn_out(part, wo_p, bo):
    p = (part[0] + part[1])[:N_NODES]
    wv = p[:, 0:64]
    z16 = p[:, 64:80]
    zc = jnp.concatenate([z16, z16, z16, z16], axis=1)
    ha = wv / (zc + 1e-6)
    return _dot(ha, wo_p) + bo


def _ffn(x, w1, b1, w2, b2):
    t = jax.nn.relu(_dot(x, w1) + b1)
    return _dot(t, w2) + b2


def _post0_body(part_r, hh0_r, wo_r, bo_r, g1_r, be1_r, w1_r, b1_r,
                w2_r, b2_r, g2_r, be2_r, wq_r, wkv_r,
                qt_o, kvt_o):
    hh = _attn_out(part_r[...], wo_r[...], bo_r[...])
    hh = hh0_r[...] + hh
    hh = _bn_fwd(hh, g1_r[...], be1_r[...])
    hh = hh + _ffn(hh, w1_r[...], b1_r[...], w2_r[...], b2_r[...])
    hh = _bn_fwd(hh, g2_r[...], be2_r[...])
    q = _dot(hh, wq_r[...])
    qt_o[...] = jnp.concatenate([q, jnp.zeros_like(q)], axis=1)
    kvt_o[...] = _dot(hh, wkv_r[...])


def _post0(part, hh0, wo_p, bo, g1, be1, w1, b1, w2, b2, g2, be2,
           wq1_t, wkv1_t):
    f = jax.ShapeDtypeStruct
    return pl.pallas_call(
        _post0_body,
        out_shape=(
            f((N_NODES, 128), jnp.float32),
            f((N_NODES, 128), jnp.float32),
        ),
    )(part, hh0, wo_p, bo, g1, be1, w1, b1, w2, b2, g2, be2,
      wq1_t, wkv1_t)


def _post1_body(part_r, wo_r, bo_r, g1_r, be1_r, w1_r, b1_r,
                w2_r, b2_r, g2_r, be2_r, mw0_r, mb0_r, mw1_r, mb1_r,
                mw2_r, mb2_r, y_o):
    hh = _attn_out(part_r[...], wo_r[...], bo_r[...])
    hh = _bn_fwd(hh, g1_r[...], be1_r[...])
    hh = _ffn(hh, w1_r[...], b1_r[...], w2_r[...], b2_r[...])
    hh = _bn_fwd(hh, g2_r[...], be2_r[...])
    hg = jnp.mean(hh, axis=0, keepdims=True)
    y = jax.nn.relu(_dot(hg, mw0_r[...]) + mb0_r[...])
    y = jax.nn.relu(_dot(y, mw1_r[...]) + mb1_r[...])
    y_o[...] = _dot(y, mw2_r[...]) + mb2_r[...]


def _post1(part, wo_p, bo, g1, be1, w1, b1, w2, b2, g2, be2,
           mw0, mb0, mw1, mb1, mw2, mb2):
    return pl.pallas_call(
        _post1_body,
        out_shape=jax.ShapeDtypeStruct((1, 3), jnp.float32),
    )(part, wo_p, bo, g1, be1, w1, b1, w2, b2, g2, be2,
      mw0, mb0, mw1, mb1, mw2, mb2)


def _row(v):
    return v.reshape(1, -1)


def kernel(h, e, edge_index, params):
    p0 = params['layer0']
    p1 = params['layer1']

    wq0 = p0['Wq'][:, _P]
    wk0 = p0['Wk'][:, _P]
    wv0 = p0['Wv'][:, _P]
    wq1 = p1['Wq'][:, _P]
    wk1 = p1['Wk'][:, _P]
    wv1 = p1['Wv'][:, _P]
    wo0 = p0['Wo'][_P, :]
    wo1 = p1['Wo'][_P, :]
    m0 = (params['We_emb'] @ p0['We'])[:, _P]
    c0 = _row((params['be_emb'] @ p0['We'])[_P])
    m1 = (params['We_emb'] @ p1['We'])[:, _P]
    c1 = _row((params['be_emb'] @ p1['We'])[_P])

    src = edge_index[0]
    dst = edge_index[1]

    wkv0 = jnp.concatenate([wk0, wv0], axis=1)
    wkv1 = jnp.concatenate([wk1, wv1], axis=1)

    hh0, qt0, kvt0 = _prep(h, params['Wh'], _row(params['bh']), wq0, wkv0)
    et0 = _eproj(e, m0, c0)

    part0 = _edge_phase(qt0, kvt0, et0, src, dst)
    et1 = _eproj(e, m1, c1)
    qt1, kvt1 = _post0(
        part0, hh0, wo0, _row(p0['bo']),
        _row(p0['bn1_g']), _row(p0['bn1_b']),
        p0['W1'], _row(p0['b1']), p0['W2'], _row(p0['b2']),
        _row(p0['bn2_g']), _row(p0['bn2_b']),
        wq1, wkv1)

    part1 = _edge_phase(qt1, kvt1, et1, src, dst)
    y = _post1(
        part1, wo1, _row(p1['bo']),
        _row(p1['bn1_g']), _row(p1['bn1_b']),
        p1['W1'], _row(p1['b1']), p1['W2'], _row(p1['b2']),
        _row(p1['bn2_g']), _row(p1['bn2_b']),
        params['mlp_W0'], _row(params['mlp_b0']),
        params['mlp_W1'], _row(params['mlp_b1']),
        params['mlp_W2'], _row(params['mlp_b2']))
    return y


# speedup vs baseline: 80.8646x; 1.1332x over previous
import functools

import jax
import jax.numpy as jnp
import numpy as np
from jax import lax
from jax.experimental import pallas as pl
from jax.experimental.pallas import tpu as pltpu
from jax.experimental.pallas import tpu_sc as plsc

N_NODES = 10000
N_EDGES = 320000
HEADS = 8
DH = 8
HID = 64

NC = 2
NS = 16
NTILES = NC * NS
EDGES_PER_TILE = N_EDGES // NTILES
EDGE_BATCH = 40
N_BATCHES = EDGES_PER_TILE // EDGE_BATCH
N_PAD = 10240
ROWS_PER_TILE = N_PAD // NS

_P = np.empty(64, np.int32)
for _h in range(HEADS):
    for _d in range(DH):
        _col = 16 * (_d // 2) + (_h if _d % 2 == 0 else 15 - _h)
        _P[_col] = _h * DH + _d


def _edge_body(qt, kvt, et, src, dst, zac,
               out_acc,
               acc, srcb, dstb, kvb, qb, eb, ob, semA, semB, semI):
    cid = lax.axis_index("c")
    sid = lax.axis_index("s")
    wid = cid * NS + sid

    r0 = sid * ROWS_PER_TILE
    pltpu.sync_copy(zac.at[pl.ds(r0, ROWS_PER_TILE)],
                    acc.at[pl.ds(r0, ROWS_PER_TILE)])

    def zrow(i, c):
        for j in range(5, 8):
            ob[i, pl.ds(j * 16, 16)] = jnp.zeros((16,), jnp.float32)
        return c

    lax.fori_loop(0, EDGE_BATCH, zrow, 0)
    plsc.subcore_barrier()

    base0 = wid * EDGES_PER_TILE

    def fire(b, buf, sem):
        base = base0 + b * EDGE_BATCH
        ci = pltpu.async_copy(src.at[pl.ds(base, EDGE_BATCH)],
                              srcb.at[buf], semI)
        cj = pltpu.async_copy(dst.at[pl.ds(base, EDGE_BATCH)],
                              dstb.at[buf], semI)
        ci.wait()
        cj.wait()
        pltpu.async_copy(kvt.at[srcb.at[buf]], kvb.at[buf], sem)
        pltpu.async_copy(qt.at[dstb.at[buf]], qb.at[buf], sem)
        pltpu.async_copy(et.at[pl.ds(base, EDGE_BATCH)], eb.at[buf], sem)

    def drain(buf, sem):
        pltpu.make_async_copy(kvt.at[pl.ds(0, EDGE_BATCH)], kvb.at[buf],
                              sem).wait()
        pltpu.make_async_copy(qt.at[pl.ds(0, EDGE_BATCH)], qb.at[buf],
                              sem).wait()
        pltpu.make_async_copy(et.at[pl.ds(0, EDGE_BATCH)], eb.at[buf],
                              sem).wait()

    def compute(buf):
        def edge(i, carry2):
            t = None
            for j in range(4):
                sl = pl.ds(j * 16, 16)
                kq = kvb[buf, i, sl] * qb[buf, i, sl] * eb[buf, i, sl]
                t = kq if t is None else t + kq
            s = t + lax.rev(t, (0,))
            s = jnp.minimum(jnp.maximum(s, -5.0), 5.0)
            w = jnp.exp(s)
            for j in range(4):
                ob[i, pl.ds(j * 16, 16)] = kvb[buf, i, pl.ds(64 + j * 16, 16)] * w
            ob[i, pl.ds(64, 16)] = w
            return carry2

        lax.fori_loop(0, EDGE_BATCH, edge, 0)
        pltpu.sync_copy(ob, acc.at[dstb.at[buf]], add=True)

    fire(0, 0, semA)

    def pair(g, carry):
        b0 = 2 * g

        @pl.when(b0 + 1 < N_BATCHES)
        def _():
            fire(b0 + 1, 1, semB)

        drain(0, semA)
        compute(0)

        @pl.when(b0 + 2 < N_BATCHES)
        def _():
            fire(b0 + 2, 0, semA)

        @pl.when(b0 + 1 < N_BATCHES)
        def _():
            drain(1, semB)
            compute(1)

        return carry

    lax.fori_loop(0, (N_BATCHES + 1) // 2, pair, 0)
    plsc.subcore_barrier()

    pltpu.sync_copy(acc.at[pl.ds(r0, ROWS_PER_TILE)],
                    out_acc.at[cid, pl.ds(r0, ROWS_PER_TILE)])


@functools.cache
def _edge_call():
    return pl.kernel(
        _edge_body,
        out_type=jax.ShapeDtypeStruct((NC, N_PAD, 128), jnp.float32),
        mesh=plsc.VectorSubcoreMesh(core_axis_name="c", subcore_axis_name="s"),
        scratch_types=[
            pltpu.VMEM_SHARED((N_PAD, 128), jnp.float32),
            pltpu.VMEM((2, EDGE_BATCH), jnp.int32),
            pltpu.VMEM((2, EDGE_BATCH), jnp.int32),
            pltpu.VMEM((2, EDGE_BATCH, 128), jnp.float32),
            pltpu.VMEM((2, EDGE_BATCH, 128), jnp.float32),
            pltpu.VMEM((2, EDGE_BATCH, 64), jnp.float32),
            pltpu.VMEM((EDGE_BATCH, 128), jnp.float32),
            pltpu.SemaphoreType.DMA,
            pltpu.SemaphoreType.DMA,
            pltpu.SemaphoreType.DMA,
        ],
    )


def _edge_phase(qt, kvt, et, src, dst):
    zac = jnp.zeros((N_PAD, 128), jnp.float32)
    return _edge_call()(qt, kvt, et, src, dst, zac)


def _dot(a, b):
    return jnp.dot(a, b, preferred_element_type=jnp.float32)


def _bn_fwd(x, g, b):
    m = jnp.mean(x, axis=0, keepdims=True)
    v = jnp.mean((x - m) ** 2, axis=0, keepdims=True)
    return g * (x - m) * jax.lax.rsqrt(v + 1e-5) + b


def _prep_body(h_r, wh_r, bh_r, wq_r, wkv_r, hh_o, qt_o, kvt_o):
    hh = _dot(h_r[...], wh_r[...]) + bh_r[...]
    hh_o[...] = hh
    q = _dot(hh, wq_r[...])
    qt_o[...] = jnp.concatenate([q, jnp.zeros_like(q)], axis=1)
    kvt_o[...] = _dot(hh, wkv_r[...])


def _prep(h, wh, bh, wq_t, wkv_t):
    f = jax.ShapeDtypeStruct
    return pl.pallas_call(
        _prep_body,
        out_shape=(
            f((N_NODES, HID), jnp.float32),
            f((N_NODES, 128), jnp.float32),
            f((N_NODES, 128), jnp.float32),
        ),
    )(h, wh, bh, wq_t, wkv_t)


_EBLK = 3200


def _eproj_body(e_r, m_r, c_r, et_o):
    et_o[...] = _dot(e_r[...], m_r[...]) + c_r[...]


def _eproj(e, m, c):
    return pl.pallas_call(
        _eproj_body,
        grid=(N_EDGES // _EBLK,),
        in_specs=[
            pl.BlockSpec((_EBLK, 16), lambda i: (i, 0)),
            pl.BlockSpec((16, 64), lambda i: (0, 0)),
            pl.BlockSpec((1, 64), lambda i: (0, 0)),
        ],
        out_specs=pl.BlockSpec((_EBLK, 64), lambda i: (i, 0)),
        out_shape=jax.ShapeDtypeStruct((N_EDGES, 64), jnp.float32),
    )(e, m, c)


def _attn_out(part, wo_p, bo):
    p = (part[0] + part[1])[:N_NODES]
    wv = p[:, 0:64]
    z16 = p[:, 64:80]
    zc = jnp.concatenate([z16, z16, z16, z16], axis=1)
    ha = wv / (zc + 1e-6)
    return _dot(ha, wo_p) + bo


def _ffn(x, w1, b1, w2, b2):
    t = jax.nn.relu(_dot(x, w1) + b1)
    return _dot(t, w2) + b2


def _post0_body(part_r, hh0_r, wo_r, bo_r, g1_r, be1_r, w1_r, b1_r,
                w2_r, b2_r, g2_r, be2_r, wq_r, wkv_r,
                qt_o, kvt_o):
    hh = _attn_out(part_r[...], wo_r[...], bo_r[...])
    hh = hh0_r[...] + hh
    hh = _bn_fwd(hh, g1_r[...], be1_r[...])
    hh = hh + _ffn(hh, w1_r[...], b1_r[...], w2_r[...], b2_r[...])
    hh = _bn_fwd(hh, g2_r[...], be2_r[...])
    q = _dot(hh, wq_r[...])
    qt_o[...] = jnp.concatenate([q, jnp.zeros_like(q)], axis=1)
    kvt_o[...] = _dot(hh, wkv_r[...])


def _post0(part, hh0, wo_p, bo, g1, be1, w1, b1, w2, b2, g2, be2,
           wq1_t, wkv1_t):
    f = jax.ShapeDtypeStruct
    return pl.pallas_call(
        _post0_body,
        out_shape=(
            f((N_NODES, 128), jnp.float32),
            f((N_NODES, 128), jnp.float32),
        ),
    )(part, hh0, wo_p, bo, g1, be1, w1, b1, w2, b2, g2, be2,
      wq1_t, wkv1_t)


def _post1_body(part_r, wo_r, bo_r, g1_r, be1_r, w1_r, b1_r,
                w2_r, b2_r, g2_r, be2_r, mw0_r, mb0_r, mw1_r, mb1_r,
                mw2_r, mb2_r, y_o):
    hh = _attn_out(part_r[...], wo_r[...], bo_r[...])
    hh = _bn_fwd(hh, g1_r[...], be1_r[...])
    hh = _ffn(hh, w1_r[...], b1_r[...], w2_r[...], b2_r[...])
    hh = _bn_fwd(hh, g2_r[...], be2_r[...])
    hg = jnp.mean(hh, axis=0, keepdims=True)
    y = jax.nn.relu(_dot(hg, mw0_r[...]) + mb0_r[...])
    y = jax.nn.relu(_dot(y, mw1_r[...]) + mb1_r[...])
    y_o[...] = _dot(y, mw2_r[...]) + mb2_r[...]


def _post1(part, wo_p, bo, g1, be1, w1, b1, w2, b2, g2, be2,
           mw0, mb0, mw1, mb1, mw2, mb2):
    return pl.pallas_call(
        _post1_body,
        out_shape=jax.ShapeDtypeStruct((1, 3), jnp.float32),
    )(part, wo_p, bo, g1, be1, w1, b1, w2, b2, g2, be2,
      mw0, mb0, mw1, mb1, mw2, mb2)


def _row(v):
    return v.reshape(1, -1)


def kernel(h, e, edge_index, params):
    p0 = params['layer0']
    p1 = params['layer1']
    scale = np.float32(1.0 / np.sqrt(DH))

    wq0 = p0['Wq'][:, _P] * scale
    wk0 = p0['Wk'][:, _P]
    wv0 = p0['Wv'][:, _P]
    wq1 = p1['Wq'][:, _P] * scale
    wk1 = p1['Wk'][:, _P]
    wv1 = p1['Wv'][:, _P]
    wo0 = p0['Wo'][_P, :]
    wo1 = p1['Wo'][_P, :]
    m0 = (params['We_emb'] @ p0['We'])[:, _P]
    c0 = _row((params['be_emb'] @ p0['We'])[_P])
    m1 = (params['We_emb'] @ p1['We'])[:, _P]
    c1 = _row((params['be_emb'] @ p1['We'])[_P])

    src = edge_index[0]
    dst = edge_index[1]

    wkv0 = jnp.concatenate([wk0, wv0], axis=1)
    wkv1 = jnp.concatenate([wk1, wv1], axis=1)

    hh0, qt0, kvt0 = _prep(h, params['Wh'], _row(params['bh']), wq0, wkv0)
    et0 = _eproj(e, m0, c0)

    part0 = _edge_phase(qt0, kvt0, et0, src, dst)
    et1 = _eproj(e, m1, c1)
    qt1, kvt1 = _post0(
        part0, hh0, wo0, _row(p0['bo']),
        _row(p0['bn1_g']), _row(p0['bn1_b']),
        p0['W1'], _row(p0['b1']), p0['W2'], _row(p0['b2']),
        _row(p0['bn2_g']), _row(p0['bn2_b']),
        wq1, wkv1)

    part1 = _edge_phase(qt1, kvt1, et1, src, dst)
    y = _post1(
        part1, wo1, _row(p1['bo']),
        _row(p1['bn1_g']), _row(p1['bn1_b']),
        p1['W1'], _row(p1['b1']), p1['W2'], _row(p1['b2']),
        _row(p1['bn2_g']), _row(p1['bn2_b']),
        params['mlp_W0'], _row(params['mlp_b0']),
        params['mlp_W1'], _row(params['mlp_b1']),
        params['mlp_W2'], _row(params['mlp_b2']))
    return y


# speedup vs baseline: 85.5926x; 1.0585x over previous
import functools

import jax
import jax.numpy as jnp
import numpy as np
from jax import lax
from jax.experimental import pallas as pl
from jax.experimental.pallas import tpu as pltpu
from jax.experimental.pallas import tpu_sc as plsc

N_NODES = 10000
N_EDGES = 320000
HEADS = 8
DH = 8
HID = 64

NC = 2
NS = 16
NTILES = NC * NS
EDGES_PER_TILE = N_EDGES // NTILES
EDGE_BATCH = 40
N_BATCHES = EDGES_PER_TILE // EDGE_BATCH
N_PAD = 10240
ROWS_PER_TILE = N_PAD // NS

_P = np.empty(64, np.int32)
for _h in range(HEADS):
    for _d in range(DH):
        _col = 16 * (_d // 2) + (_h if _d % 2 == 0 else 15 - _h)
        _P[_col] = _h * DH + _d


def _edge_body(qt, kvt, et, src, dst, zac,
               out_acc,
               acc, srcb, dstb, dsts, kvb, qb, eb, ob,
               semA, semB, semI0, semI1, semS0, semS1):
    cid = lax.axis_index("c")
    sid = lax.axis_index("s")
    wid = cid * NS + sid

    r0 = sid * ROWS_PER_TILE
    pltpu.sync_copy(zac.at[pl.ds(r0, ROWS_PER_TILE)],
                    acc.at[pl.ds(r0, ROWS_PER_TILE)])

    def zrow(i, c):
        for buf in range(2):
            for j in range(5, 8):
                ob[buf, i, pl.ds(j * 16, 16)] = jnp.zeros((16,), jnp.float32)
        return c

    lax.fori_loop(0, EDGE_BATCH, zrow, 0)
    plsc.subcore_barrier()

    base0 = wid * EDGES_PER_TILE

    def fire_idx_src(b, buf, semI):
        base = base0 + b * EDGE_BATCH
        pltpu.async_copy(src.at[pl.ds(base, EDGE_BATCH)], srcb.at[buf], semI)

    def fire_idx_dst(b, buf, semI):
        base = base0 + b * EDGE_BATCH
        pltpu.async_copy(dst.at[pl.ds(base, EDGE_BATCH)], dstb.at[buf], semI)

    def fire_gathers(b, buf, sem, semI):
        base = base0 + b * EDGE_BATCH
        pltpu.make_async_copy(src.at[pl.ds(0, EDGE_BATCH)], srcb.at[buf],
                              semI).wait()
        pltpu.make_async_copy(dst.at[pl.ds(0, EDGE_BATCH)], dstb.at[buf],
                              semI).wait()
        pltpu.async_copy(kvt.at[srcb.at[buf]], kvb.at[buf], sem)
        pltpu.async_copy(qt.at[dstb.at[buf]], qb.at[buf], sem)
        pltpu.async_copy(et.at[pl.ds(base, EDGE_BATCH)], eb.at[buf], sem)

    def drain_gathers(buf, sem):
        pltpu.make_async_copy(kvt.at[pl.ds(0, EDGE_BATCH)], kvb.at[buf],
                              sem).wait()
        pltpu.make_async_copy(qt.at[pl.ds(0, EDGE_BATCH)], qb.at[buf],
                              sem).wait()
        pltpu.make_async_copy(et.at[pl.ds(0, EDGE_BATCH)], eb.at[buf],
                              sem).wait()

    def drain_scatter(buf, semS):
        pltpu.make_async_copy(ob.at[buf], acc.at[pl.ds(0, EDGE_BATCH)],
                              semS).wait()

    def compute(b, buf, semS, semI, nb2_ok):
        @pl.when(b >= 2)
        def _():
            drain_scatter(buf, semS)

        def edge(i, carry2):
            t = None
            for j in range(4):
                sl = pl.ds(j * 16, 16)
                kq = kvb[buf, i, sl] * qb[buf, i, sl] * eb[buf, i, sl]
                t = kq if t is None else t + kq
            s = t + lax.rev(t, (0,))
            s = jnp.minimum(jnp.maximum(s, -5.0), 5.0)
            w = jnp.exp(s)
            for j in range(4):
                ob[buf, i, pl.ds(j * 16, 16)] = (
                    kvb[buf, i, pl.ds(64 + j * 16, 16)] * w)
            ob[buf, i, pl.ds(64, 16)] = w
            return carry2

        lax.fori_loop(0, EDGE_BATCH, edge, 0)
        for off in (0, 16, 24):
            dsts[buf, pl.ds(off, 16)] = dstb[buf, pl.ds(off, 16)]

        @pl.when(nb2_ok)
        def _():
            fire_idx_dst(b + 2, buf, semI)

        pltpu.async_copy(ob.at[buf], acc.at[dsts.at[buf]], semS, add=True)

    fire_idx_src(0, 0, semI0)
    fire_idx_dst(0, 0, semI0)
    fire_idx_src(1, 1, semI1)
    fire_idx_dst(1, 1, semI1)
    fire_gathers(0, 0, semA, semI0)

    def pair(g, carry):
        b0 = 2 * g

        @pl.when(b0 + 1 < N_BATCHES)
        def _():
            fire_gathers(b0 + 1, 1, semB, semI1)

        drain_gathers(0, semA)

        @pl.when(b0 + 2 < N_BATCHES)
        def _():
            fire_idx_src(b0 + 2, 0, semI0)

        compute(b0, 0, semS0, semI0, b0 + 2 < N_BATCHES)

        @pl.when(b0 + 2 < N_BATCHES)
        def _():
            fire_gathers(b0 + 2, 0, semA, semI0)

        @pl.when(b0 + 1 < N_BATCHES)
        def _():
            drain_gathers(1, semB)

            @pl.when(b0 + 3 < N_BATCHES)
            def _():
                fire_idx_src(b0 + 3, 1, semI1)

            compute(b0 + 1, 1, semS1, semI1, b0 + 3 < N_BATCHES)

        return carry

    lax.fori_loop(0, (N_BATCHES + 1) // 2, pair, 0)
    drain_scatter(0, semS0)
    drain_scatter(1, semS1)
    plsc.subcore_barrier()

    pltpu.sync_copy(acc.at[pl.ds(r0, ROWS_PER_TILE)],
                    out_acc.at[cid, pl.ds(r0, ROWS_PER_TILE)])


@functools.cache
def _edge_call():
    return pl.kernel(
        _edge_body,
        out_type=jax.ShapeDtypeStruct((NC, N_PAD, 128), jnp.float32),
        mesh=plsc.VectorSubcoreMesh(core_axis_name="c", subcore_axis_name="s"),
        scratch_types=[
            pltpu.VMEM_SHARED((N_PAD, 128), jnp.float32),
            pltpu.VMEM((2, EDGE_BATCH), jnp.int32),
            pltpu.VMEM((2, EDGE_BATCH), jnp.int32),
            pltpu.VMEM((2, EDGE_BATCH), jnp.int32),
            pltpu.VMEM((2, EDGE_BATCH, 128), jnp.float32),
            pltpu.VMEM((2, EDGE_BATCH, 128), jnp.float32),
            pltpu.VMEM((2, EDGE_BATCH, 64), jnp.float32),
            pltpu.VMEM((2, EDGE_BATCH, 128), jnp.float32),
            pltpu.SemaphoreType.DMA,
            pltpu.SemaphoreType.DMA,
            pltpu.SemaphoreType.DMA,
            pltpu.SemaphoreType.DMA,
            pltpu.SemaphoreType.DMA,
            pltpu.SemaphoreType.DMA,
        ],
    )


def _edge_phase(qt, kvt, et, src, dst):
    zac = jnp.zeros((N_PAD, 128), jnp.float32)
    return _edge_call()(qt, kvt, et, src, dst, zac)


def _dot(a, b):
    return jnp.dot(a, b, preferred_element_type=jnp.float32)


def _bn_fwd(x, g, b):
    m = jnp.mean(x, axis=0, keepdims=True)
    v = jnp.mean((x - m) ** 2, axis=0, keepdims=True)
    return g * (x - m) * jax.lax.rsqrt(v + 1e-5) + b


def _prep_body(h_r, wh_r, bh_r, wq_r, wkv_r, hh_o, qt_o, kvt_o):
    hh = _dot(h_r[...], wh_r[...]) + bh_r[...]
    hh_o[...] = hh
    q = _dot(hh, wq_r[...])
    qt_o[...] = jnp.concatenate([q, jnp.zeros_like(q)], axis=1)
    kvt_o[...] = _dot(hh, wkv_r[...])


def _prep(h, wh, bh, wq_t, wkv_t):
    f = jax.ShapeDtypeStruct
    return pl.pallas_call(
        _prep_body,
        out_shape=(
            f((N_NODES, HID), jnp.float32),
            f((N_NODES, 128), jnp.float32),
            f((N_NODES, 128), jnp.float32),
        ),
    )(h, wh, bh, wq_t, wkv_t)


_EBLK = 3200


def _eproj_body(e_r, m_r, c_r, et_o):
    et_o[...] = _dot(e_r[...], m_r[...]) + c_r[...]


def _eproj(e, m, c):
    return pl.pallas_call(
        _eproj_body,
        grid=(N_EDGES // _EBLK,),
        in_specs=[
            pl.BlockSpec((_EBLK, 16), lambda i: (i, 0)),
            pl.BlockSpec((16, 64), lambda i: (0, 0)),
            pl.BlockSpec((1, 64), lambda i: (0, 0)),
        ],
        out_specs=pl.BlockSpec((_EBLK, 64), lambda i: (i, 0)),
        out_shape=jax.ShapeDtypeStruct((N_EDGES, 64), jnp.float32),
    )(e, m, c)


def _attn_out(part, wo_p, bo):
    p = (part[0] + part[1])[:N_NODES]
    wv = p[:, 0:64]
    z16 = p[:, 64:80]
    zc = jnp.concatenate([z16, z16, z16, z16], axis=1)
    ha = wv / (zc + 1e-6)
    return _dot(ha, wo_p) + bo


def _ffn(x, w1, b1, w2, b2):
    t = jax.nn.relu(_dot(x, w1) + b1)
    return _dot(t, w2) + b2


def _post0_body(part_r, hh0_r, wo_r, bo_r, g1_r, be1_r, w1_r, b1_r,
                w2_r, b2_r, g2_r, be2_r, wq_r, wkv_r,
                qt_o, kvt_o):
    hh = _attn_out(part_r[...], wo_r[...], bo_r[...])
    hh = hh0_r[...] + hh
    hh = _bn_fwd(hh, g1_r[...], be1_r[...])
    hh = hh + _ffn(hh, w1_r[...], b1_r[...], w2_r[...], b2_r[...])
    hh = _bn_fwd(hh, g2_r[...], be2_r[...])
    q = _dot(hh, wq_r[...])
    qt_o[...] = jnp.concatenate([q, jnp.zeros_like(q)], axis=1)
    kvt_o[...] = _dot(hh, wkv_r[...])


def _post0(part, hh0, wo_p, bo, g1, be1, w1, b1, w2, b2, g2, be2,
           wq1_t, wkv1_t):
    f = jax.ShapeDtypeStruct
    return pl.pallas_call(
        _post0_body,
        out_shape=(
            f((N_NODES, 128), jnp.float32),
            f((N_NODES, 128), jnp.float32),
        ),
    )(part, hh0, wo_p, bo, g1, be1, w1, b1, w2, b2, g2, be2,
      wq1_t, wkv1_t)


def _post1_body(part_r, wo_r, bo_r, g1_r, be1_r, w1_r, b1_r,
                w2_r, b2_r, g2_r, be2_r, mw0_r, mb0_r, mw1_r, mb1_r,
                mw2_r, mb2_r, y_o):
    hh = _attn_out(part_r[...], wo_r[...], bo_r[...])
    hh = _bn_fwd(hh, g1_r[...], be1_r[...])
    hh = _ffn(hh, w1_r[...], b1_r[...], w2_r[...], b2_r[...])
    hh = _bn_fwd(hh, g2_r[...], be2_r[...])
    hg = jnp.mean(hh, axis=0, keepdims=True)
    y = jax.nn.relu(_dot(hg, mw0_r[...]) + mb0_r[...])
    y = jax.nn.relu(_dot(y, mw1_r[...]) + mb1_r[...])
    y_o[...] = _dot(y, mw2_r[...]) + mb2_r[...]


def _post1(part, wo_p, bo, g1, be1, w1, b1, w2, b2, g2, be2,
           mw0, mb0, mw1, mb1, mw2, mb2):
    return pl.pallas_call(
        _post1_body,
        out_shape=jax.ShapeDtypeStruct((1, 3), jnp.float32),
    )(part, wo_p, bo, g1, be1, w1, b1, w2, b2, g2, be2,
      mw0, mb0, mw1, mb1, mw2, mb2)


def _row(v):
    return v.reshape(1, -1)


def kernel(h, e, edge_index, params):
    p0 = params['layer0']
    p1 = params['layer1']
    scale = np.float32(1.0 / np.sqrt(DH))

    wq0 = p0['Wq'][:, _P] * scale
    wk0 = p0['Wk'][:, _P]
    wv0 = p0['Wv'][:, _P]
    wq1 = p1['Wq'][:, _P] * scale
    wk1 = p1['Wk'][:, _P]
    wv1 = p1['Wv'][:, _P]
    wo0 = p0['Wo'][_P, :]
    wo1 = p1['Wo'][_P, :]
    m0 = (params['We_emb'] @ p0['We'])[:, _P]
    c0 = _row((params['be_emb'] @ p0['We'])[_P])
    m1 = (params['We_emb'] @ p1['We'])[:, _P]
    c1 = _row((params['be_emb'] @ p1['We'])[_P])

    src = edge_index[0]
    dst = edge_index[1]

    wkv0 = jnp.concatenate([wk0, wv0], axis=1)
    wkv1 = jnp.concatenate([wk1, wv1], axis=1)

    hh0, qt0, kvt0 = _prep(h, params['Wh'], _row(params['bh']), wq0, wkv0)
    et0 = _eproj(e, m0, c0)

    part0 = _edge_phase(qt0, kvt0, et0, src, dst)
    et1 = _eproj(e, m1, c1)
    qt1, kvt1 = _post0(
        part0, hh0, wo0, _row(p0['bo']),
        _row(p0['bn1_g']), _row(p0['bn1_b']),
        p0['W1'], _row(p0['b1']), p0['W2'], _row(p0['b2']),
        _row(p0['bn2_g']), _row(p0['bn2_b']),
        wq1, wkv1)

    part1 = _edge_phase(qt1, kvt1, et1, src, dst)
    y = _post1(
        part1, wo1, _row(p1['bo']),
        _row(p1['bn1_g']), _row(p1['bn1_b']),
        p1['W1'], _row(p1['b1']), p1['W2'], _row(p1['b2']),
        _row(p1['bn2_g']), _row(p1['bn2_b']),
        params['mlp_W0'], _row(params['mlp_b0']),
        params['mlp_W1'], _row(params['mlp_b1']),
        params['mlp_W2'], _row(params['mlp_b2']))
    return y
